# baseline (speedup 1.0000x reference)
import functools

import jax
import jax.numpy as jnp
from jax import lax
from jax.experimental import pallas as pl
from jax.experimental.pallas import tpu as pltpu
from jax.experimental.pallas import tpu_sc as plsc

_ID2CAT = tuple(int(2 + (i * 97) % 398) for i in range(120))
_MAX_ATT = 397
_NUM_ATTR = 120
_D = 768
_K = 512
_SIDE = 32
_SCALE = 32.0 / 512.0


_D16 = _D // 16


_CH = 64
_NG = 4
_GV = _D16 // _NG


def _sc_pool_call(x2d, meta_flat):
    info = plsc.get_sparse_core_info()
    nc, ns = info.num_cores, info.num_subcores
    nw = nc * ns
    bpw = _K // nw
    mesh = plsc.VectorSubcoreMesh(core_axis_name="c", subcore_axis_name="s")

    @functools.partial(
        pl.kernel,
        mesh=mesh,
        out_type=jax.ShapeDtypeStruct((_K, _D), jnp.float32),
        scratch_types=[
            pltpu.VMEM((bpw * 16,), jnp.int32),
            pltpu.VMEM((_CH,), jnp.int32),
            pltpu.VMEM((_CH,), jnp.int32),
            pltpu.VMEM((_CH, _D), jnp.float32),
            pltpu.VMEM((_CH, _D), jnp.float32),
            pltpu.VMEM((_D,), jnp.float32),
            pltpu.SemaphoreType.DMA,
            pltpu.SemaphoreType.DMA,
        ],
    )
    def k(x_hbm, meta_hbm, out_hbm, meta_v, idx0, idx1, rows0, rows1,
          acc_v, sem0, sem1):
        wid = lax.axis_index("s") * nc + lax.axis_index("c")
        base_box = wid * bpw
        pltpu.sync_copy(meta_hbm.at[pl.ds(base_box * 16, bpw * 16)], meta_v)
        lanes = lax.iota(jnp.int32, 16)
        cnt = meta_v[pl.ds(0, 16)][5]

        for d in range(_D16):
            acc_v[pl.ds(d * 16, 16)] = jnp.zeros((16,), jnp.float32)

        def zero_box(j, c):
            outrow = meta_v[pl.ds(j * 16, 16)][4]
            pltpu.sync_copy(acc_v, out_hbm.at[outrow])
            return c

        lax.fori_loop(cnt, bpw, zero_box, 0)

        def fire(slot, chunk, idx_v, rows_v, sem):
            srow = meta_v[pl.ds(slot * 16, 16)]
            basef, nf, ncolf, magicf = srow[0], srow[1], srow[2], srow[3]
            for u in range(_CH // 16):
                t = jnp.minimum(chunk * _CH + u * 16 + lanes, nf - 1)
                yy = lax.shift_right_logical(t * magicf, 16)
                idx_v[pl.ds(u * 16, 16)] = basef + yy * _SIDE + (t - yy * ncolf)
            pltpu.async_copy(x_hbm.at[idx_v], rows_v, sem)

        def advance(slot, chunk):
            srow = meta_v[pl.ds(slot * 16, 16)]
            nch = lax.shift_right_logical(srow[1] + (_CH - 1), 6)
            over = (chunk + 1) >= nch
            return (jnp.where(over, slot + 1, slot),
                    jnp.where(over, 0, chunk + 1))

        def reduce_rows(rows_v, pc, nrows, last, outrow):
            for g in range(_NG):
                first = pc == 0
                regs = [
                    jnp.where(first, jnp.full((16,), -jnp.inf, jnp.float32),
                              acc_v[pl.ds((g * _GV + v) * 16, 16)])
                    for v in range(_GV)
                ]

                def row_step(r, regs):
                    return tuple(
                        jnp.maximum(regs[v],
                                    rows_v[r, pl.ds((g * _GV + v) * 16, 16)])
                        for v in range(_GV))

                regs = lax.fori_loop(0, nrows, row_step, tuple(regs))
                for v in range(_GV):
                    acc_v[pl.ds((g * _GV + v) * 16, 16)] = regs[v]

            @pl.when(last)
            def _():
                pltpu.sync_copy(acc_v, out_hbm.at[outrow])

        tchunks = meta_v[pl.ds(0, 16)][6]

        @pl.when(cnt > 0)
        def _():
            fire(0, 0, idx0, rows0, sem0)

        fi0, fc0 = advance(0, 0)

        def body(q, carry):
            pi, pc, fi, fc = carry
            parity = jnp.bitwise_and(q, 1)

            @pl.when((fi < cnt) & (parity == 0))
            def _():
                fire(fi, fc, idx1, rows1, sem1)

            @pl.when((fi < cnt) & (parity == 1))
            def _():
                fire(fi, fc, idx0, rows0, sem0)

            prow = meta_v[pl.ds(pi * 16, 16)]
            np_ = prow[1]
            outrow = prow[4]
            nchp = lax.shift_right_logical(np_ + (_CH - 1), 6)
            nrows = jnp.minimum(np_ - pc * _CH, _CH)
            last = (pc + 1) >= nchp

            @pl.when(parity == 0)
            def _():
                pltpu.make_async_copy(x_hbm.at[pl.ds(0, _CH)], rows0,
                                      sem0).wait()
                reduce_rows(rows0, pc, nrows, last, outrow)

            @pl.when(parity == 1)
            def _():
                pltpu.make_async_copy(x_hbm.at[pl.ds(0, _CH)], rows1,
                                      sem1).wait()
                reduce_rows(rows1, pc, nrows, last, outrow)

            fi2, fc2 = advance(fi, fc)
            fi3 = jnp.where(fi < cnt, fi2, fi)
            fc3 = jnp.where(fi < cnt, fc2, fc)
            pi2, pc2 = advance(pi, pc)
            return (pi2, pc2, fi3, fc3)

        lax.fori_loop(0, tchunks, body,
                      (jnp.int32(0), jnp.int32(0),
                       fi0.astype(jnp.int32), fc0.astype(jnp.int32)))

    return k(x2d, meta_flat)


_KS = 1352


def _ff_body(p_ref, wff_ref, bff_ref, g_ref, be_ref, wd_ref, bd_ref,
             perm_ref, h_ref, hs_ref, disr_ref):
    h0 = jnp.dot(p_ref[:], wff_ref[:], preferred_element_type=jnp.float32)
    h0 = h0 + bff_ref[:]
    mu = jnp.mean(h0, axis=-1, keepdims=True)
    var = jnp.mean((h0 - mu) ** 2, axis=-1, keepdims=True)
    hn = (h0 - mu) / jnp.sqrt(var + 1e-5) * g_ref[:] + be_ref[:]
    h = hn * 0.5 * (1.0 + lax.erf(hn / jnp.sqrt(jnp.float32(2.0))))
    h_ref[:] = h
    disr_ref[:] = jnp.dot(h, wd_ref[:], preferred_element_type=jnp.float32) + bd_ref[:]
    sel = (lax.broadcasted_iota(jnp.int32, (_KS, _K), 1) == perm_ref[:]
           ).astype(jnp.float32)
    hs_ref[:] = jnp.dot(sel, h, preferred_element_type=jnp.float32)


def _ff(pooled, W_ff, b_ff, ln_g, ln_b, W_disr, b_disr, spos):
    return pl.pallas_call(
        _ff_body,
        out_shape=(jax.ShapeDtypeStruct((_K, _D), jnp.float32),
                   jax.ShapeDtypeStruct((_KS, _D), jnp.float32),
                   jax.ShapeDtypeStruct((_K, 1), jnp.float32)),
    )(pooled, W_ff, b_ff.reshape(1, _D), ln_g.reshape(1, _D),
      ln_b.reshape(1, _D), W_disr, b_disr.reshape(1, 1),
      spos.reshape(_KS, 1))


def _heads_body(poffs_ref, cats_ref, hs_ref, w_ref, bh_ref, out_ref):
    e = pl.program_id(0)
    start = pl.multiple_of(poffs_ref[e], 8)
    nch = (poffs_ref[e + 1] - poffs_ref[e]) // 8
    cat = cats_ref[e]
    colmask = lax.broadcasted_iota(jnp.int32, (8, _MAX_ATT), 1) < cat

    @pl.when(e == 0)
    def _():
        out_ref[...] = jnp.zeros((_KS, _MAX_ATT), jnp.float32)

    def chunk(c, carry):
        o = start + c * 8
        rows = hs_ref[pl.ds(o, 8), :]
        prod = jnp.dot(rows, w_ref[0], preferred_element_type=jnp.float32)
        out_ref[pl.ds(o, 8), :] = jnp.where(colmask, prod + bh_ref[0, 0], 0.0)
        return carry

    lax.fori_loop(0, nch, chunk, 0)


def _heads(poffs, cats, h_sorted, W_heads, b_heads):
    grid_spec = pltpu.PrefetchScalarGridSpec(
        num_scalar_prefetch=2,
        grid=(_NUM_ATTR,),
        in_specs=[
            pl.BlockSpec((_KS, _D), lambda e, o, c: (0, 0)),
            pl.BlockSpec((1, _D, _MAX_ATT), lambda e, o, c: (e, 0, 0)),
            pl.BlockSpec((1, 1, _MAX_ATT), lambda e, o, c: (e, 0, 0)),
        ],
        out_specs=pl.BlockSpec((_KS, _MAX_ATT), lambda e, o, c: (0, 0)),
    )
    return pl.pallas_call(
        _heads_body,
        grid_spec=grid_spec,
        out_shape=jax.ShapeDtypeStruct((_KS, _MAX_ATT), jnp.float32),
    )(poffs, cats, h_sorted, W_heads,
      b_heads.reshape(_NUM_ATTR, 1, _MAX_ATT))


def _unsort_body(sposr_ref, ls_ref, out_ref):
    sel = (lax.broadcasted_iota(jnp.int32, (_K, _KS), 0) == sposr_ref[:]
           ).astype(jnp.float32)
    out_ref[:] = jnp.dot(sel, ls_ref[:], preferred_element_type=jnp.float32)


def _unsort(spos, logits_sorted):
    return pl.pallas_call(
        _unsort_body,
        out_shape=jax.ShapeDtypeStruct((_K, _MAX_ATT), jnp.float32),
    )(spos.reshape(1, _KS), logits_sorted)


def kernel(x, boxes, box_labels, W_ff, b_ff, ln_g, ln_b, W_disr, b_disr,
           W_heads, b_heads):
    q = jnp.round(boxes[:, 1:5].astype(jnp.float32) * _SCALE).astype(jnp.int32)
    x1, y1, x2, y2 = q[:, 0], q[:, 1], q[:, 2], q[:, 3]
    roi_w = jnp.maximum(x2 - x1 + 1, 1)
    roi_h = jnp.maximum(y2 - y1 + 1, 1)
    hs = jnp.clip(y1, 0, _SIDE)
    he = jnp.clip(y1 + roi_h, 0, _SIDE)
    ws = jnp.clip(x1, 0, _SIDE)
    we = jnp.clip(x1 + roi_w, 0, _SIDE)
    b = boxes[:, 0].astype(jnp.int32)
    nrows = he - hs
    ncols = we - ws
    base = b * (_SIDE * _SIDE) + hs * _SIDE + ws
    n = nrows * ncols
    magic = (65536 + jnp.maximum(ncols, 1) - 1) // jnp.maximum(ncols, 1)
    vg = (n > 0).reshape(32, 16)
    order = jnp.argsort(~vg, axis=1, stable=True)
    gidx = (jnp.arange(32, dtype=jnp.int32)[:, None] * 16
            + order.astype(jnp.int32)).reshape(-1)
    cnt = jnp.repeat(vg.sum(axis=1).astype(jnp.int32), 16)
    tchunks = jnp.repeat(
        ((n + 63) // 64).reshape(32, 16).sum(axis=1).astype(jnp.int32), 16)
    meta = jnp.stack(
        [base[gidx], n[gidx], ncols[gidx], magic[gidx], gidx, cnt, tchunks]
        + [jnp.zeros_like(gidx)] * 9, axis=1)

    pooled = _sc_pool_call(x.reshape(-1, _D), meta.reshape(-1))

    labels = box_labels.astype(jnp.int32)
    perm = jnp.argsort(labels).astype(jnp.int32)
    counts = jnp.zeros((_NUM_ATTR,), jnp.int32).at[labels].add(1)
    offs = jnp.concatenate([jnp.zeros((1,), jnp.int32),
                            jnp.cumsum(counts).astype(jnp.int32)])
    pcounts = ((counts + 7) // 8) * 8
    poffs = jnp.concatenate([jnp.zeros((1,), jnp.int32),
                             jnp.cumsum(pcounts).astype(jnp.int32)])
    cats = jnp.asarray(_ID2CAT, dtype=jnp.int32)
    labels_sorted = jnp.sort(labels)
    dest = poffs[labels_sorted] + jnp.arange(_K, dtype=jnp.int32) \
        - offs[labels_sorted]
    spos = jnp.full((_KS,), -1, jnp.int32).at[dest].set(perm)

    h, h_sorted, disr_logits = _ff(pooled, W_ff, b_ff, ln_g, ln_b,
                                   W_disr, b_disr, spos)
    logits_sorted = _heads(poffs, cats, h_sorted, W_heads, b_heads)
    logits = _unsort(spos, logits_sorted)
    return (h, logits, disr_logits)


# speedup vs baseline: 1.0365x; 1.0365x over previous
import functools

import jax
import jax.numpy as jnp
from jax import lax
from jax.experimental import pallas as pl
from jax.experimental.pallas import tpu as pltpu
from jax.experimental.pallas import tpu_sc as plsc

_ID2CAT = tuple(int(2 + (i * 97) % 398) for i in range(120))
_MAX_ATT = 397
_NUM_ATTR = 120
_D = 768
_K = 512
_SIDE = 32
_SCALE = 32.0 / 512.0


_D16 = _D // 16


_CH = 64
_NG = 4
_GV = _D16 // _NG


def _sc_pool_call(x2d, meta_flat):
    info = plsc.get_sparse_core_info()
    nc, ns = info.num_cores, info.num_subcores
    nw = nc * ns
    bpw = _K // nw
    mesh = plsc.VectorSubcoreMesh(core_axis_name="c", subcore_axis_name="s")

    @functools.partial(
        pl.kernel,
        mesh=mesh,
        out_type=jax.ShapeDtypeStruct((_K, _D), jnp.float32),
        scratch_types=[
            pltpu.VMEM((bpw * 16,), jnp.int32),
            pltpu.VMEM((_CH,), jnp.int32),
            pltpu.VMEM((_CH,), jnp.int32),
            pltpu.VMEM((_CH, _D), jnp.float32),
            pltpu.VMEM((_CH, _D), jnp.float32),
            pltpu.VMEM((_D,), jnp.float32),
            pltpu.SemaphoreType.DMA,
            pltpu.SemaphoreType.DMA,
        ],
    )
    def k(x_hbm, meta_hbm, out_hbm, meta_v, idx0, idx1, rows0, rows1,
          acc_v, sem0, sem1):
        wid = lax.axis_index("s") * nc + lax.axis_index("c")
        base_box = wid * bpw
        pltpu.sync_copy(meta_hbm.at[pl.ds(base_box * 16, bpw * 16)], meta_v)
        lanes = lax.iota(jnp.int32, 16)

        def fire(slot, chunk, idx_v, rows_v, sem):
            srow = meta_v[pl.ds(slot * 16, 16)]
            basef, nf, ncolf, magicf = srow[0], srow[1], srow[2], srow[3]
            for u in range(_CH // 16):
                t = jnp.minimum(chunk * _CH + u * 16 + lanes, nf - 1)
                yy = lax.shift_right_logical(t * magicf, 16)
                idx_v[pl.ds(u * 16, 16)] = basef + yy * _SIDE + (t - yy * ncolf)
            pltpu.async_copy(x_hbm.at[idx_v], rows_v, sem)

        def advance(slot, chunk):
            srow = meta_v[pl.ds(slot * 16, 16)]
            nch = lax.shift_right_logical(srow[1] + (_CH - 1), 6)
            over = (chunk + 1) >= nch
            return (jnp.where(over, slot + 1, slot),
                    jnp.where(over, 0, chunk + 1))

        def reduce_rows(rows_v, pc, nrows, last, valid, outrow):
            for g in range(_NG):
                first = pc == 0
                regs = [
                    jnp.where(first, jnp.full((16,), -jnp.inf, jnp.float32),
                              acc_v[pl.ds((g * _GV + v) * 16, 16)])
                    for v in range(_GV)
                ]

                def row_step(r, regs):
                    return tuple(
                        jnp.maximum(regs[v],
                                    rows_v[r, pl.ds((g * _GV + v) * 16, 16)])
                        for v in range(_GV))

                regs = lax.fori_loop(0, nrows, row_step, tuple(regs))
                for v in range(_GV):
                    acc_v[pl.ds((g * _GV + v) * 16, 16)] = jnp.where(
                        valid > 0, regs[v], jnp.zeros((16,), jnp.float32))

            @pl.when(last)
            def _():
                pltpu.sync_copy(acc_v, out_hbm.at[outrow])

        tchunks = meta_v[pl.ds(0, 16)][5]
        fire(0, 0, idx0, rows0, sem0)
        fi0, fc0 = advance(0, 0)

        def body(q, carry):
            pi, pc, fi, fc = carry
            parity = jnp.bitwise_and(q, 1)

            @pl.when((fi < bpw) & (parity == 0))
            def _():
                fire(fi, fc, idx1, rows1, sem1)

            @pl.when((fi < bpw) & (parity == 1))
            def _():
                fire(fi, fc, idx0, rows0, sem0)

            prow = meta_v[pl.ds(pi * 16, 16)]
            np_ = prow[1]
            valid = prow[4]
            outrow = base_box + pi
            nchp = lax.shift_right_logical(np_ + (_CH - 1), 6)
            nrows = jnp.minimum(np_ - pc * _CH, _CH)
            last = (pc + 1) >= nchp

            @pl.when(parity == 0)
            def _():
                pltpu.make_async_copy(x_hbm.at[pl.ds(0, _CH)], rows0,
                                      sem0).wait()
                reduce_rows(rows0, pc, nrows, last, valid, outrow)

            @pl.when(parity == 1)
            def _():
                pltpu.make_async_copy(x_hbm.at[pl.ds(0, _CH)], rows1,
                                      sem1).wait()
                reduce_rows(rows1, pc, nrows, last, valid, outrow)

            fi2, fc2 = advance(fi, fc)
            fi3 = jnp.where(fi < bpw, fi2, fi)
            fc3 = jnp.where(fi < bpw, fc2, fc)
            pi2, pc2 = advance(pi, pc)
            return (pi2, pc2, fi3, fc3)

        lax.fori_loop(0, tchunks, body,
                      (jnp.int32(0), jnp.int32(0),
                       fi0.astype(jnp.int32), fc0.astype(jnp.int32)))

    return k(x2d, meta_flat)


_KS = 1352


def _ff_body(p_ref, wff_ref, bff_ref, g_ref, be_ref, wd_ref, bd_ref,
             perm_ref, h_ref, hs_ref, disr_ref):
    h0 = jnp.dot(p_ref[:], wff_ref[:], preferred_element_type=jnp.float32)
    h0 = h0 + bff_ref[:]
    mu = jnp.mean(h0, axis=-1, keepdims=True)
    var = jnp.mean((h0 - mu) ** 2, axis=-1, keepdims=True)
    hn = (h0 - mu) / jnp.sqrt(var + 1e-5) * g_ref[:] + be_ref[:]
    h = hn * 0.5 * (1.0 + lax.erf(hn / jnp.sqrt(jnp.float32(2.0))))
    h_ref[:] = h
    disr_ref[:] = jnp.dot(h, wd_ref[:], preferred_element_type=jnp.float32) + bd_ref[:]
    sel = (lax.broadcasted_iota(jnp.int32, (_KS, _K), 1) == perm_ref[:]
           ).astype(jnp.float32)
    hs_ref[:] = jnp.dot(sel, h, preferred_element_type=jnp.float32)


def _ff(pooled, W_ff, b_ff, ln_g, ln_b, W_disr, b_disr, spos):
    return pl.pallas_call(
        _ff_body,
        out_shape=(jax.ShapeDtypeStruct((_K, _D), jnp.float32),
                   jax.ShapeDtypeStruct((_KS, _D), jnp.float32),
                   jax.ShapeDtypeStruct((_K, 1), jnp.float32)),
    )(pooled, W_ff, b_ff.reshape(1, _D), ln_g.reshape(1, _D),
      ln_b.reshape(1, _D), W_disr, b_disr.reshape(1, 1),
      spos.reshape(_KS, 1))


def _heads_body(poffs_ref, cats_ref, hs_ref, w_ref, bh_ref, out_ref):
    e = pl.program_id(0)
    start = pl.multiple_of(poffs_ref[e], 8)
    nch = (poffs_ref[e + 1] - poffs_ref[e]) // 8
    cat = cats_ref[e]
    colmask = lax.broadcasted_iota(jnp.int32, (8, _MAX_ATT), 1) < cat

    @pl.when(e == 0)
    def _():
        out_ref[...] = jnp.zeros((_KS, _MAX_ATT), jnp.float32)

    def chunk(c, carry):
        o = start + c * 8
        rows = hs_ref[pl.ds(o, 8), :]
        prod = jnp.dot(rows, w_ref[0], preferred_element_type=jnp.float32)
        out_ref[pl.ds(o, 8), :] = jnp.where(colmask, prod + bh_ref[0, 0], 0.0)
        return carry

    lax.fori_loop(0, nch, chunk, 0)


def _heads(poffs, cats, h_sorted, W_heads, b_heads):
    grid_spec = pltpu.PrefetchScalarGridSpec(
        num_scalar_prefetch=2,
        grid=(_NUM_ATTR,),
        in_specs=[
            pl.BlockSpec((_KS, _D), lambda e, o, c: (0, 0)),
            pl.BlockSpec((1, _D, _MAX_ATT), lambda e, o, c: (e, 0, 0)),
            pl.BlockSpec((1, 1, _MAX_ATT), lambda e, o, c: (e, 0, 0)),
        ],
        out_specs=pl.BlockSpec((_KS, _MAX_ATT), lambda e, o, c: (0, 0)),
    )
    return pl.pallas_call(
        _heads_body,
        grid_spec=grid_spec,
        out_shape=jax.ShapeDtypeStruct((_KS, _MAX_ATT), jnp.float32),
    )(poffs, cats, h_sorted, W_heads,
      b_heads.reshape(_NUM_ATTR, 1, _MAX_ATT))


def _unsort_body(sposr_ref, ls_ref, out_ref):
    sel = (lax.broadcasted_iota(jnp.int32, (_K, _KS), 0) == sposr_ref[:]
           ).astype(jnp.float32)
    out_ref[:] = jnp.dot(sel, ls_ref[:], preferred_element_type=jnp.float32)


def _unsort(spos, logits_sorted):
    return pl.pallas_call(
        _unsort_body,
        out_shape=jax.ShapeDtypeStruct((_K, _MAX_ATT), jnp.float32),
    )(spos.reshape(1, _KS), logits_sorted)


def kernel(x, boxes, box_labels, W_ff, b_ff, ln_g, ln_b, W_disr, b_disr,
           W_heads, b_heads):
    q = jnp.round(boxes[:, 1:5].astype(jnp.float32) * _SCALE).astype(jnp.int32)
    x1, y1, x2, y2 = q[:, 0], q[:, 1], q[:, 2], q[:, 3]
    roi_w = jnp.maximum(x2 - x1 + 1, 1)
    roi_h = jnp.maximum(y2 - y1 + 1, 1)
    hs = jnp.clip(y1, 0, _SIDE)
    he = jnp.clip(y1 + roi_h, 0, _SIDE)
    ws = jnp.clip(x1, 0, _SIDE)
    we = jnp.clip(x1 + roi_w, 0, _SIDE)
    b = boxes[:, 0].astype(jnp.int32)
    nrows = he - hs
    ncols = we - ws
    base = b * (_SIDE * _SIDE) + hs * _SIDE + ws
    n = nrows * ncols
    magic = (65536 + jnp.maximum(ncols, 1) - 1) // jnp.maximum(ncols, 1)
    valid = (n > 0).astype(jnp.int32)
    n_eff = jnp.maximum(n, 1)
    ncol_eff = jnp.maximum(ncols, 1)
    base_eff = jnp.where(n > 0, base, 0)
    tchunks = jnp.repeat(
        ((n_eff + 63) // 64).reshape(32, 16).sum(axis=1).astype(jnp.int32),
        16)
    meta = jnp.stack(
        [base_eff, n_eff, ncol_eff, magic, valid, tchunks]
        + [jnp.zeros_like(n)] * 10, axis=1)

    pooled = _sc_pool_call(x.reshape(-1, _D), meta.reshape(-1))

    labels = box_labels.astype(jnp.int32)
    perm = jnp.argsort(labels).astype(jnp.int32)
    counts = jnp.zeros((_NUM_ATTR,), jnp.int32).at[labels].add(1)
    offs = jnp.concatenate([jnp.zeros((1,), jnp.int32),
                            jnp.cumsum(counts).astype(jnp.int32)])
    pcounts = ((counts + 7) // 8) * 8
    poffs = jnp.concatenate([jnp.zeros((1,), jnp.int32),
                             jnp.cumsum(pcounts).astype(jnp.int32)])
    cats = jnp.asarray(_ID2CAT, dtype=jnp.int32)
    labels_sorted = jnp.sort(labels)
    dest = poffs[labels_sorted] + jnp.arange(_K, dtype=jnp.int32) \
        - offs[labels_sorted]
    spos = jnp.full((_KS,), -1, jnp.int32).at[dest].set(perm)

    h, h_sorted, disr_logits = _ff(pooled, W_ff, b_ff, ln_g, ln_b,
                                   W_disr, b_disr, spos)
    logits_sorted = _heads(poffs, cats, h_sorted, W_heads, b_heads)
    logits = _unsort(spos, logits_sorted)
    return (h, logits, disr_logits)


# speedup vs baseline: 1.1606x; 1.1198x over previous
import functools

import jax
import jax.numpy as jnp
from jax import lax
from jax.experimental import pallas as pl
from jax.experimental.pallas import tpu as pltpu
from jax.experimental.pallas import tpu_sc as plsc

_ID2CAT = tuple(int(2 + (i * 97) % 398) for i in range(120))
_MAX_ATT = 397
_NUM_ATTR = 120
_D = 768
_K = 512
_SIDE = 32
_SCALE = 32.0 / 512.0


_D16 = _D // 16


_CH = 64
_NG = 4
_GV = _D16 // _NG


def _sc_pool_call(x2d, meta_flat):
    info = plsc.get_sparse_core_info()
    nc, ns = info.num_cores, info.num_subcores
    nw = nc * ns
    bpw = _K // nw
    mesh = plsc.VectorSubcoreMesh(core_axis_name="c", subcore_axis_name="s")

    @functools.partial(
        pl.kernel,
        mesh=mesh,
        out_type=jax.ShapeDtypeStruct((_K, _D), jnp.float32),
        scratch_types=[
            pltpu.VMEM((bpw * 16,), jnp.int32),
            pltpu.VMEM((_CH,), jnp.int32),
            pltpu.VMEM((_CH,), jnp.int32),
            pltpu.VMEM((_CH, _D), jnp.float32),
            pltpu.VMEM((_CH, _D), jnp.float32),
            pltpu.VMEM((_D,), jnp.float32),
            pltpu.SemaphoreType.DMA,
            pltpu.SemaphoreType.DMA,
        ],
    )
    def k(x_hbm, meta_hbm, out_hbm, meta_v, idx0, idx1, rows0, rows1,
          acc_v, sem0, sem1):
        wid = lax.axis_index("s") * nc + lax.axis_index("c")
        base_box = wid * bpw
        pltpu.sync_copy(meta_hbm.at[pl.ds(base_box * 16, bpw * 16)], meta_v)
        lanes = lax.iota(jnp.int32, 16)

        def fire(slot, chunk, idx_v, rows_v, sem):
            srow = meta_v[pl.ds(slot * 16, 16)]
            basef, nf, ncolf, magicf = srow[0], srow[1], srow[2], srow[3]
            for u in range(_CH // 16):
                t = jnp.minimum(chunk * _CH + u * 16 + lanes, nf - 1)
                yy = lax.shift_right_logical(t * magicf, 16)
                idx_v[pl.ds(u * 16, 16)] = basef + yy * _SIDE + (t - yy * ncolf)
            pltpu.async_copy(x_hbm.at[idx_v], rows_v, sem)

        def advance(slot, chunk):
            srow = meta_v[pl.ds(slot * 16, 16)]
            nch = lax.shift_right_logical(srow[1] + (_CH - 1), 6)
            over = (chunk + 1) >= nch
            return (jnp.where(over, slot + 1, slot),
                    jnp.where(over, 0, chunk + 1))

        def reduce_rows(rows_v, pc, nrows, last, valid, outrow):
            for g in range(_NG):
                first = pc == 0
                regs = [
                    jnp.where(first, jnp.full((16,), -jnp.inf, jnp.float32),
                              acc_v[pl.ds((g * _GV + v) * 16, 16)])
                    for v in range(_GV)
                ]

                def row_step(r, regs):
                    return tuple(
                        jnp.maximum(regs[v],
                                    rows_v[r, pl.ds((g * _GV + v) * 16, 16)])
                        for v in range(_GV))

                regs = lax.fori_loop(0, nrows, row_step, tuple(regs))
                for v in range(_GV):
                    acc_v[pl.ds((g * _GV + v) * 16, 16)] = jnp.where(
                        valid > 0, regs[v], jnp.zeros((16,), jnp.float32))

            @pl.when(last)
            def _():
                pltpu.sync_copy(acc_v, out_hbm.at[outrow])

        tchunks = meta_v[pl.ds(0, 16)][5]
        fire(0, 0, idx0, rows0, sem0)
        fi0, fc0 = advance(0, 0)

        def body(q, carry):
            pi, pc, fi, fc = carry
            parity = jnp.bitwise_and(q, 1)

            @pl.when((fi < bpw) & (parity == 0))
            def _():
                fire(fi, fc, idx1, rows1, sem1)

            @pl.when((fi < bpw) & (parity == 1))
            def _():
                fire(fi, fc, idx0, rows0, sem0)

            prow = meta_v[pl.ds(pi * 16, 16)]
            np_ = prow[1]
            valid = prow[4]
            outrow = base_box + pi
            nchp = lax.shift_right_logical(np_ + (_CH - 1), 6)
            nrows = jnp.minimum(np_ - pc * _CH, _CH)
            last = (pc + 1) >= nchp

            @pl.when(parity == 0)
            def _():
                pltpu.make_async_copy(x_hbm.at[pl.ds(0, _CH)], rows0,
                                      sem0).wait()
                reduce_rows(rows0, pc, nrows, last, valid, outrow)

            @pl.when(parity == 1)
            def _():
                pltpu.make_async_copy(x_hbm.at[pl.ds(0, _CH)], rows1,
                                      sem1).wait()
                reduce_rows(rows1, pc, nrows, last, valid, outrow)

            fi2, fc2 = advance(fi, fc)
            fi3 = jnp.where(fi < bpw, fi2, fi)
            fc3 = jnp.where(fi < bpw, fc2, fc)
            pi2, pc2 = advance(pi, pc)
            return (pi2, pc2, fi3, fc3)

        lax.fori_loop(0, tchunks, body,
                      (jnp.int32(0), jnp.int32(0),
                       fi0.astype(jnp.int32), fc0.astype(jnp.int32)))

    return k(x2d, meta_flat)


_KS = 1352


def _ff_body(p_ref, wff_ref, bff_ref, g_ref, be_ref, wd_ref, bd_ref,
             perm_ref, h_ref, hs_ref, disr_ref):
    h0 = jnp.dot(p_ref[:], wff_ref[:], preferred_element_type=jnp.float32)
    h0 = h0 + bff_ref[:]
    mu = jnp.mean(h0, axis=-1, keepdims=True)
    var = jnp.mean((h0 - mu) ** 2, axis=-1, keepdims=True)
    hn = (h0 - mu) / jnp.sqrt(var + 1e-5) * g_ref[:] + be_ref[:]
    h = hn * 0.5 * (1.0 + lax.erf(hn / jnp.sqrt(jnp.float32(2.0))))
    h_ref[:] = h
    disr_ref[:] = jnp.dot(h, wd_ref[:], preferred_element_type=jnp.float32) + bd_ref[:]
    sel = (lax.broadcasted_iota(jnp.int32, (_KS, _K), 1) == perm_ref[:]
           ).astype(jnp.float32)
    hs_ref[:] = jnp.dot(sel, h, preferred_element_type=jnp.float32)


def _ff(pooled, W_ff, b_ff, ln_g, ln_b, W_disr, b_disr, spos):
    return pl.pallas_call(
        _ff_body,
        out_shape=(jax.ShapeDtypeStruct((_K, _D), jnp.float32),
                   jax.ShapeDtypeStruct((_KS, _D), jnp.float32),
                   jax.ShapeDtypeStruct((_K, 1), jnp.float32)),
    )(pooled, W_ff, b_ff.reshape(1, _D), ln_g.reshape(1, _D),
      ln_b.reshape(1, _D), W_disr, b_disr.reshape(1, 1),
      spos.reshape(_KS, 1))


_LPS = 4


def _heads_body(poffs_ref, cats_ref, hs_ref, w_ref, bh_ref, out_ref):
    g = pl.program_id(0)

    @pl.when(g == 0)
    def _():
        out_ref[...] = jnp.zeros((_KS, _MAX_ATT), jnp.float32)

    for el in range(_LPS):
        e = g * _LPS + el
        start = pl.multiple_of(poffs_ref[e], 8)
        nch = (poffs_ref[e + 1] - poffs_ref[e]) // 8
        cat = cats_ref[e]
        colmask = lax.broadcasted_iota(jnp.int32, (8, _MAX_ATT), 1) < cat

        def chunk(c, carry):
            o = start + c * 8
            rows = hs_ref[pl.ds(o, 8), :]
            prod = jnp.dot(rows, w_ref[el],
                           preferred_element_type=jnp.float32)
            out_ref[pl.ds(o, 8), :] = jnp.where(
                colmask, prod + bh_ref[el, 0], 0.0)
            return carry

        lax.fori_loop(0, nch, chunk, 0)


def _heads(poffs, cats, h_sorted, W_heads, b_heads):
    grid_spec = pltpu.PrefetchScalarGridSpec(
        num_scalar_prefetch=2,
        grid=(_NUM_ATTR // _LPS,),
        in_specs=[
            pl.BlockSpec((_KS, _D), lambda g, o, c: (0, 0)),
            pl.BlockSpec((_LPS, _D, _MAX_ATT), lambda g, o, c: (g, 0, 0)),
            pl.BlockSpec((_LPS, 1, _MAX_ATT), lambda g, o, c: (g, 0, 0)),
        ],
        out_specs=pl.BlockSpec((_KS, _MAX_ATT), lambda g, o, c: (0, 0)),
    )
    return pl.pallas_call(
        _heads_body,
        grid_spec=grid_spec,
        out_shape=jax.ShapeDtypeStruct((_KS, _MAX_ATT), jnp.float32),
    )(poffs, cats, h_sorted, W_heads,
      b_heads.reshape(_NUM_ATTR, 1, _MAX_ATT))


def _unsort_body(sposr_ref, ls_ref, out_ref):
    sel = (lax.broadcasted_iota(jnp.int32, (_K, _KS), 0) == sposr_ref[:]
           ).astype(jnp.float32)
    out_ref[:] = jnp.dot(sel, ls_ref[:], preferred_element_type=jnp.float32)


def _unsort(spos, logits_sorted):
    return pl.pallas_call(
        _unsort_body,
        out_shape=jax.ShapeDtypeStruct((_K, _MAX_ATT), jnp.float32),
    )(spos.reshape(1, _KS), logits_sorted)


def kernel(x, boxes, box_labels, W_ff, b_ff, ln_g, ln_b, W_disr, b_disr,
           W_heads, b_heads):
    q = jnp.round(boxes[:, 1:5].astype(jnp.float32) * _SCALE).astype(jnp.int32)
    x1, y1, x2, y2 = q[:, 0], q[:, 1], q[:, 2], q[:, 3]
    roi_w = jnp.maximum(x2 - x1 + 1, 1)
    roi_h = jnp.maximum(y2 - y1 + 1, 1)
    hs = jnp.clip(y1, 0, _SIDE)
    he = jnp.clip(y1 + roi_h, 0, _SIDE)
    ws = jnp.clip(x1, 0, _SIDE)
    we = jnp.clip(x1 + roi_w, 0, _SIDE)
    b = boxes[:, 0].astype(jnp.int32)
    nrows = he - hs
    ncols = we - ws
    base = b * (_SIDE * _SIDE) + hs * _SIDE + ws
    n = nrows * ncols
    magic = (65536 + jnp.maximum(ncols, 1) - 1) // jnp.maximum(ncols, 1)
    valid = (n > 0).astype(jnp.int32)
    n_eff = jnp.maximum(n, 1)
    ncol_eff = jnp.maximum(ncols, 1)
    base_eff = jnp.where(n > 0, base, 0)
    tchunks = jnp.repeat(
        ((n_eff + 63) // 64).reshape(32, 16).sum(axis=1).astype(jnp.int32),
        16)
    meta = jnp.stack(
        [base_eff, n_eff, ncol_eff, magic, valid, tchunks]
        + [jnp.zeros_like(n)] * 10, axis=1)

    pooled = _sc_pool_call(x.reshape(-1, _D), meta.reshape(-1))

    labels = box_labels.astype(jnp.int32)
    perm = jnp.argsort(labels).astype(jnp.int32)
    counts = jnp.zeros((_NUM_ATTR,), jnp.int32).at[labels].add(1)
    offs = jnp.concatenate([jnp.zeros((1,), jnp.int32),
                            jnp.cumsum(counts).astype(jnp.int32)])
    pcounts = ((counts + 7) // 8) * 8
    poffs = jnp.concatenate([jnp.zeros((1,), jnp.int32),
                             jnp.cumsum(pcounts).astype(jnp.int32)])
    cats = jnp.asarray(_ID2CAT, dtype=jnp.int32)
    labels_sorted = jnp.sort(labels)
    dest = poffs[labels_sorted] + jnp.arange(_K, dtype=jnp.int32) \
        - offs[labels_sorted]
    spos = jnp.full((_KS,), -1, jnp.int32).at[dest].set(perm)

    h, h_sorted, disr_logits = _ff(pooled, W_ff, b_ff, ln_g, ln_b,
                                   W_disr, b_disr, spos)
    logits_sorted = _heads(poffs, cats, h_sorted, W_heads, b_heads)
    logits = _unsort(spos, logits_sorted)
    return (h, logits, disr_logits)


# speedup vs baseline: 1.1633x; 1.0023x over previous
import functools

import jax
import jax.numpy as jnp
from jax import lax
from jax.experimental import pallas as pl
from jax.experimental.pallas import tpu as pltpu
from jax.experimental.pallas import tpu_sc as plsc

_ID2CAT = tuple(int(2 + (i * 97) % 398) for i in range(120))
_MAX_ATT = 397
_NUM_ATTR = 120
_D = 768
_K = 512
_SIDE = 32
_SCALE = 32.0 / 512.0


_D16 = _D // 16


_CH = 64
_NG = 4
_GV = _D16 // _NG


def _sc_pool_call(x2d, meta_flat):
    info = plsc.get_sparse_core_info()
    nc, ns = info.num_cores, info.num_subcores
    nw = nc * ns
    bpw = _K // nw
    mesh = plsc.VectorSubcoreMesh(core_axis_name="c", subcore_axis_name="s")

    @functools.partial(
        pl.kernel,
        mesh=mesh,
        out_type=jax.ShapeDtypeStruct((_K, _D), jnp.float32),
        scratch_types=[
            pltpu.VMEM((bpw * 16,), jnp.int32),
            pltpu.VMEM((_CH,), jnp.int32),
            pltpu.VMEM((_CH,), jnp.int32),
            pltpu.VMEM((_CH, _D), jnp.float32),
            pltpu.VMEM((_CH, _D), jnp.float32),
            pltpu.VMEM((_D,), jnp.float32),
            pltpu.SemaphoreType.DMA,
            pltpu.SemaphoreType.DMA,
        ],
    )
    def k(x_hbm, meta_hbm, out_hbm, meta_v, idx0, idx1, rows0, rows1,
          acc_v, sem0, sem1):
        wid = lax.axis_index("s") * nc + lax.axis_index("c")
        base_box = wid * bpw
        pltpu.sync_copy(meta_hbm.at[pl.ds(base_box * 16, bpw * 16)], meta_v)
        lanes = lax.iota(jnp.int32, 16)

        def fire(slot, chunk, idx_v, rows_v, sem):
            srow = meta_v[pl.ds(slot * 16, 16)]
            basef, nf, ncolf, magicf = srow[0], srow[1], srow[2], srow[3]
            for u in range(_CH // 16):
                t = jnp.minimum(chunk * _CH + u * 16 + lanes, nf - 1)
                yy = lax.shift_right_logical(t * magicf, 16)
                idx_v[pl.ds(u * 16, 16)] = basef + yy * _SIDE + (t - yy * ncolf)
            pltpu.async_copy(x_hbm.at[idx_v], rows_v, sem)

        def advance(slot, chunk):
            srow = meta_v[pl.ds(slot * 16, 16)]
            nch = lax.shift_right_logical(srow[1] + (_CH - 1), 6)
            over = (chunk + 1) >= nch
            return (jnp.where(over, slot + 1, slot),
                    jnp.where(over, 0, chunk + 1))

        def reduce_rows(rows_v, pc, nrows, last, valid, outrow):
            for g in range(_NG):
                first = pc == 0
                regs = [
                    jnp.where(first, jnp.full((16,), -jnp.inf, jnp.float32),
                              acc_v[pl.ds((g * _GV + v) * 16, 16)])
                    for v in range(_GV)
                ]

                def row_step(r, regs):
                    return tuple(
                        jnp.maximum(regs[v],
                                    rows_v[r, pl.ds((g * _GV + v) * 16, 16)])
                        for v in range(_GV))

                regs = lax.fori_loop(0, nrows, row_step, tuple(regs))
                for v in range(_GV):
                    acc_v[pl.ds((g * _GV + v) * 16, 16)] = jnp.where(
                        valid > 0, regs[v], jnp.zeros((16,), jnp.float32))

            @pl.when(last)
            def _():
                pltpu.sync_copy(acc_v, out_hbm.at[outrow])

        tchunks = meta_v[pl.ds(0, 16)][5]
        fire(0, 0, idx0, rows0, sem0)
        fi0, fc0 = advance(0, 0)

        def body(q, carry):
            pi, pc, fi, fc = carry
            parity = jnp.bitwise_and(q, 1)

            @pl.when((fi < bpw) & (parity == 0))
            def _():
                fire(fi, fc, idx1, rows1, sem1)

            @pl.when((fi < bpw) & (parity == 1))
            def _():
                fire(fi, fc, idx0, rows0, sem0)

            prow = meta_v[pl.ds(pi * 16, 16)]
            np_ = prow[1]
            valid = prow[4]
            outrow = base_box + pi
            nchp = lax.shift_right_logical(np_ + (_CH - 1), 6)
            nrows = jnp.minimum(np_ - pc * _CH, _CH)
            last = (pc + 1) >= nchp

            @pl.when(parity == 0)
            def _():
                pltpu.make_async_copy(x_hbm.at[pl.ds(0, _CH)], rows0,
                                      sem0).wait()
                reduce_rows(rows0, pc, nrows, last, valid, outrow)

            @pl.when(parity == 1)
            def _():
                pltpu.make_async_copy(x_hbm.at[pl.ds(0, _CH)], rows1,
                                      sem1).wait()
                reduce_rows(rows1, pc, nrows, last, valid, outrow)

            fi2, fc2 = advance(fi, fc)
            fi3 = jnp.where(fi < bpw, fi2, fi)
            fc3 = jnp.where(fi < bpw, fc2, fc)
            pi2, pc2 = advance(pi, pc)
            return (pi2, pc2, fi3, fc3)

        lax.fori_loop(0, tchunks, body,
                      (jnp.int32(0), jnp.int32(0),
                       fi0.astype(jnp.int32), fc0.astype(jnp.int32)))

    return k(x2d, meta_flat)


_KS = 1352


def _ff_body(p_ref, wff_ref, bff_ref, g_ref, be_ref, wd_ref, bd_ref,
             perm_ref, h_ref, hs_ref, disr_ref):
    h0 = jnp.dot(p_ref[:], wff_ref[:], preferred_element_type=jnp.float32)
    h0 = h0 + bff_ref[:]
    mu = jnp.mean(h0, axis=-1, keepdims=True)
    var = jnp.mean((h0 - mu) ** 2, axis=-1, keepdims=True)
    hn = (h0 - mu) / jnp.sqrt(var + 1e-5) * g_ref[:] + be_ref[:]
    h = hn * 0.5 * (1.0 + lax.erf(hn / jnp.sqrt(jnp.float32(2.0))))
    h_ref[:] = h
    disr_ref[:] = jnp.dot(h, wd_ref[:], preferred_element_type=jnp.float32) + bd_ref[:]
    sel = (lax.broadcasted_iota(jnp.int32, (_KS, _K), 1) == perm_ref[:]
           ).astype(jnp.float32)
    hs_ref[:] = jnp.dot(sel, h, preferred_element_type=jnp.float32)


def _ff(pooled, W_ff, b_ff, ln_g, ln_b, W_disr, b_disr, spos):
    return pl.pallas_call(
        _ff_body,
        out_shape=(jax.ShapeDtypeStruct((_K, _D), jnp.float32),
                   jax.ShapeDtypeStruct((_KS, _D), jnp.float32),
                   jax.ShapeDtypeStruct((_K, 1), jnp.float32)),
    )(pooled, W_ff, b_ff.reshape(1, _D), ln_g.reshape(1, _D),
      ln_b.reshape(1, _D), W_disr, b_disr.reshape(1, 1),
      spos.reshape(_KS, 1))


_LPS = 8


def _heads_body(poffs_ref, cats_ref, hs_ref, w_ref, bh_ref, out_ref):
    g = pl.program_id(0)

    @pl.when(g == 0)
    def _():
        out_ref[...] = jnp.zeros((_KS, _MAX_ATT), jnp.float32)

    for el in range(_LPS):
        e = g * _LPS + el
        start = pl.multiple_of(poffs_ref[e], 8)
        nch = (poffs_ref[e + 1] - poffs_ref[e]) // 8
        cat = cats_ref[e]
        colmask = lax.broadcasted_iota(jnp.int32, (8, _MAX_ATT), 1) < cat

        def chunk(c, carry):
            o = start + c * 8
            rows = hs_ref[pl.ds(o, 8), :]
            prod = jnp.dot(rows, w_ref[el],
                           preferred_element_type=jnp.float32)
            out_ref[pl.ds(o, 8), :] = jnp.where(
                colmask, prod + bh_ref[el, 0], 0.0)
            return carry

        lax.fori_loop(0, nch, chunk, 0)


def _heads(poffs, cats, h_sorted, W_heads, b_heads):
    grid_spec = pltpu.PrefetchScalarGridSpec(
        num_scalar_prefetch=2,
        grid=(_NUM_ATTR // _LPS,),
        in_specs=[
            pl.BlockSpec((_KS, _D), lambda g, o, c: (0, 0)),
            pl.BlockSpec((_LPS, _D, _MAX_ATT), lambda g, o, c: (g, 0, 0)),
            pl.BlockSpec((_LPS, 1, _MAX_ATT), lambda g, o, c: (g, 0, 0)),
        ],
        out_specs=pl.BlockSpec((_KS, _MAX_ATT), lambda g, o, c: (0, 0)),
    )
    return pl.pallas_call(
        _heads_body,
        grid_spec=grid_spec,
        out_shape=jax.ShapeDtypeStruct((_KS, _MAX_ATT), jnp.float32),
    )(poffs, cats, h_sorted, W_heads,
      b_heads.reshape(_NUM_ATTR, 1, _MAX_ATT))


def _unsort_body(sposr_ref, ls_ref, out_ref):
    sel = (lax.broadcasted_iota(jnp.int32, (_K, _KS), 0) == sposr_ref[:]
           ).astype(jnp.float32)
    out_ref[:] = jnp.dot(sel, ls_ref[:], preferred_element_type=jnp.float32)


def _unsort(spos, logits_sorted):
    return pl.pallas_call(
        _unsort_body,
        out_shape=jax.ShapeDtypeStruct((_K, _MAX_ATT), jnp.float32),
    )(spos.reshape(1, _KS), logits_sorted)


def kernel(x, boxes, box_labels, W_ff, b_ff, ln_g, ln_b, W_disr, b_disr,
           W_heads, b_heads):
    q = jnp.round(boxes[:, 1:5].astype(jnp.float32) * _SCALE).astype(jnp.int32)
    x1, y1, x2, y2 = q[:, 0], q[:, 1], q[:, 2], q[:, 3]
    roi_w = jnp.maximum(x2 - x1 + 1, 1)
    roi_h = jnp.maximum(y2 - y1 + 1, 1)
    hs = jnp.clip(y1, 0, _SIDE)
    he = jnp.clip(y1 + roi_h, 0, _SIDE)
    ws = jnp.clip(x1, 0, _SIDE)
    we = jnp.clip(x1 + roi_w, 0, _SIDE)
    b = boxes[:, 0].astype(jnp.int32)
    nrows = he - hs
    ncols = we - ws
    base = b * (_SIDE * _SIDE) + hs * _SIDE + ws
    n = nrows * ncols
    magic = (65536 + jnp.maximum(ncols, 1) - 1) // jnp.maximum(ncols, 1)
    valid = (n > 0).astype(jnp.int32)
    n_eff = jnp.maximum(n, 1)
    ncol_eff = jnp.maximum(ncols, 1)
    base_eff = jnp.where(n > 0, base, 0)
    tchunks = jnp.repeat(
        ((n_eff + 63) // 64).reshape(32, 16).sum(axis=1).astype(jnp.int32),
        16)
    meta = jnp.stack(
        [base_eff, n_eff, ncol_eff, magic, valid, tchunks]
        + [jnp.zeros_like(n)] * 10, axis=1)

    pooled = _sc_pool_call(x.reshape(-1, _D), meta.reshape(-1))

    labels = box_labels.astype(jnp.int32)
    perm = jnp.argsort(labels).astype(jnp.int32)
    counts = jnp.zeros((_NUM_ATTR,), jnp.int32).at[labels].add(1)
    offs = jnp.concatenate([jnp.zeros((1,), jnp.int32),
                            jnp.cumsum(counts).astype(jnp.int32)])
    pcounts = ((counts + 7) // 8) * 8
    poffs = jnp.concatenate([jnp.zeros((1,), jnp.int32),
                             jnp.cumsum(pcounts).astype(jnp.int32)])
    cats = jnp.asarray(_ID2CAT, dtype=jnp.int32)
    labels_sorted = jnp.sort(labels)
    dest = poffs[labels_sorted] + jnp.arange(_K, dtype=jnp.int32) \
        - offs[labels_sorted]
    spos = jnp.full((_KS,), -1, jnp.int32).at[dest].set(perm)

    h, h_sorted, disr_logits = _ff(pooled, W_ff, b_ff, ln_g, ln_b,
                                   W_disr, b_disr, spos)
    logits_sorted = _heads(poffs, cats, h_sorted, W_heads, b_heads)
    logits = _unsort(spos, logits_sorted)
    return (h, logits, disr_logits)


# speedup vs baseline: 1.1692x; 1.0050x over previous
import functools

import jax
import jax.numpy as jnp
from jax import lax
from jax.experimental import pallas as pl
from jax.experimental.pallas import tpu as pltpu
from jax.experimental.pallas import tpu_sc as plsc

_ID2CAT = tuple(int(2 + (i * 97) % 398) for i in range(120))
_MAX_ATT = 397
_NUM_ATTR = 120
_D = 768
_K = 512
_SIDE = 32
_SCALE = 32.0 / 512.0


_D16 = _D // 16


_CH = 64
_NG = 4
_GV = _D16 // _NG


def _sc_pool_call(x2d, meta_flat):
    info = plsc.get_sparse_core_info()
    nc, ns = info.num_cores, info.num_subcores
    nw = nc * ns
    bpw = _K // nw
    mesh = plsc.VectorSubcoreMesh(core_axis_name="c", subcore_axis_name="s")

    @functools.partial(
        pl.kernel,
        mesh=mesh,
        out_type=jax.ShapeDtypeStruct((_K, _D), jnp.float32),
        scratch_types=[
            pltpu.VMEM((bpw * 16,), jnp.int32),
            pltpu.VMEM((_CH,), jnp.int32),
            pltpu.VMEM((_CH,), jnp.int32),
            pltpu.VMEM((_CH, _D), jnp.float32),
            pltpu.VMEM((_CH, _D), jnp.float32),
            pltpu.VMEM((_D,), jnp.float32),
            pltpu.SemaphoreType.DMA,
            pltpu.SemaphoreType.DMA,
        ],
    )
    def k(x_hbm, meta_hbm, out_hbm, meta_v, idx0, idx1, rows0, rows1,
          acc_v, sem0, sem1):
        wid = lax.axis_index("s") * nc + lax.axis_index("c")
        base_box = wid * bpw
        pltpu.sync_copy(meta_hbm.at[pl.ds(base_box * 16, bpw * 16)], meta_v)
        lanes = lax.iota(jnp.int32, 16)

        def fire(slot, chunk, idx_v, rows_v, sem):
            srow = meta_v[pl.ds(slot * 16, 16)]
            basef, nf, ncolf, magicf = srow[0], srow[1], srow[2], srow[3]
            for u in range(_CH // 16):
                t = jnp.minimum(chunk * _CH + u * 16 + lanes, nf - 1)
                yy = lax.shift_right_logical(t * magicf, 16)
                idx_v[pl.ds(u * 16, 16)] = basef + yy * _SIDE + (t - yy * ncolf)
            pltpu.async_copy(x_hbm.at[idx_v], rows_v, sem)

        def advance(slot, chunk):
            srow = meta_v[pl.ds(slot * 16, 16)]
            nch = lax.shift_right_logical(srow[1] + (_CH - 1), 6)
            over = (chunk + 1) >= nch
            return (jnp.where(over, slot + 1, slot),
                    jnp.where(over, 0, chunk + 1))

        def reduce_rows(rows_v, pc, nrows, last, valid, outrow):
            for g in range(_NG):
                first = pc == 0
                regs = [
                    jnp.where(first, jnp.full((16,), -jnp.inf, jnp.float32),
                              acc_v[pl.ds((g * _GV + v) * 16, 16)])
                    for v in range(_GV)
                ]

                def row_step(r, regs):
                    return tuple(
                        jnp.maximum(regs[v],
                                    rows_v[r, pl.ds((g * _GV + v) * 16, 16)])
                        for v in range(_GV))

                regs = lax.fori_loop(0, nrows, row_step, tuple(regs))
                for v in range(_GV):
                    acc_v[pl.ds((g * _GV + v) * 16, 16)] = jnp.where(
                        valid > 0, regs[v], jnp.zeros((16,), jnp.float32))

            @pl.when(last)
            def _():
                pltpu.sync_copy(acc_v, out_hbm.at[outrow])

        tchunks = meta_v[pl.ds(0, 16)][5]
        fire(0, 0, idx0, rows0, sem0)
        fi0, fc0 = advance(0, 0)

        def body(q, carry):
            pi, pc, fi, fc = carry
            parity = jnp.bitwise_and(q, 1)

            @pl.when((fi < bpw) & (parity == 0))
            def _():
                fire(fi, fc, idx1, rows1, sem1)

            @pl.when((fi < bpw) & (parity == 1))
            def _():
                fire(fi, fc, idx0, rows0, sem0)

            prow = meta_v[pl.ds(pi * 16, 16)]
            np_ = prow[1]
            valid = prow[4]
            outrow = base_box + pi
            nchp = lax.shift_right_logical(np_ + (_CH - 1), 6)
            nrows = jnp.minimum(np_ - pc * _CH, _CH)
            last = (pc + 1) >= nchp

            @pl.when(parity == 0)
            def _():
                pltpu.make_async_copy(x_hbm.at[pl.ds(0, _CH)], rows0,
                                      sem0).wait()
                reduce_rows(rows0, pc, nrows, last, valid, outrow)

            @pl.when(parity == 1)
            def _():
                pltpu.make_async_copy(x_hbm.at[pl.ds(0, _CH)], rows1,
                                      sem1).wait()
                reduce_rows(rows1, pc, nrows, last, valid, outrow)

            fi2, fc2 = advance(fi, fc)
            fi3 = jnp.where(fi < bpw, fi2, fi)
            fc3 = jnp.where(fi < bpw, fc2, fc)
            pi2, pc2 = advance(pi, pc)
            return (pi2, pc2, fi3, fc3)

        lax.fori_loop(0, tchunks, body,
                      (jnp.int32(0), jnp.int32(0),
                       fi0.astype(jnp.int32), fc0.astype(jnp.int32)))

    return k(x2d, meta_flat)


_KS = 1360


def _ff_body(p_ref, wff_ref, bff_ref, g_ref, be_ref, wd_ref, bd_ref,
             perm_ref, h_ref, hs_ref, disr_ref):
    h0 = jnp.dot(p_ref[:], wff_ref[:], preferred_element_type=jnp.float32)
    h0 = h0 + bff_ref[:]
    mu = jnp.mean(h0, axis=-1, keepdims=True)
    var = jnp.mean((h0 - mu) ** 2, axis=-1, keepdims=True)
    hn = (h0 - mu) / jnp.sqrt(var + 1e-5) * g_ref[:] + be_ref[:]
    h = hn * 0.5 * (1.0 + lax.erf(hn / jnp.sqrt(jnp.float32(2.0))))
    h_ref[:] = h
    disr_ref[:] = jnp.dot(h, wd_ref[:], preferred_element_type=jnp.float32) + bd_ref[:]
    sel = (lax.broadcasted_iota(jnp.int32, (_KS, _K), 1) == perm_ref[:]
           ).astype(jnp.float32)
    hs_ref[:] = jnp.dot(sel, h, preferred_element_type=jnp.float32)


def _ff(pooled, W_ff, b_ff, ln_g, ln_b, W_disr, b_disr, spos):
    return pl.pallas_call(
        _ff_body,
        out_shape=(jax.ShapeDtypeStruct((_K, _D), jnp.float32),
                   jax.ShapeDtypeStruct((_KS, _D), jnp.float32),
                   jax.ShapeDtypeStruct((_K, 1), jnp.float32)),
    )(pooled, W_ff, b_ff.reshape(1, _D), ln_g.reshape(1, _D),
      ln_b.reshape(1, _D), W_disr, b_disr.reshape(1, 1),
      spos.reshape(_KS, 1))


_LPS = 8


def _heads_body(poffs_ref, cats_ref, hs_ref, w_ref, bh_ref, out_ref):
    g = pl.program_id(0)

    @pl.when(g == 0)
    def _():
        out_ref[...] = jnp.zeros((_KS, _MAX_ATT), jnp.float32)

    def one_chunk(el, o):
        cat = cats_ref[g * _LPS + el]
        colmask = lax.broadcasted_iota(jnp.int32, (8, _MAX_ATT), 1) < cat
        rows = hs_ref[pl.ds(o, 8), :]
        prod = jnp.dot(rows, w_ref[el], preferred_element_type=jnp.float32)
        out_ref[pl.ds(o, 8), :] = jnp.where(
            colmask, prod + bh_ref[el, 0], 0.0)

    for el in range(_LPS):
        one_chunk(el, pl.multiple_of(poffs_ref[g * _LPS + el], 8))

    for el in range(_LPS):
        e = g * _LPS + el
        start = pl.multiple_of(poffs_ref[e], 8)
        nch = (poffs_ref[e + 1] - poffs_ref[e]) // 8

        def chunk(c, carry):
            one_chunk(el, start + c * 8)
            return carry

        lax.fori_loop(1, nch, chunk, 0)


def _heads(poffs, cats, h_sorted, W_heads, b_heads):
    grid_spec = pltpu.PrefetchScalarGridSpec(
        num_scalar_prefetch=2,
        grid=(_NUM_ATTR // _LPS,),
        in_specs=[
            pl.BlockSpec((_KS, _D), lambda g, o, c: (0, 0)),
            pl.BlockSpec((_LPS, _D, _MAX_ATT), lambda g, o, c: (g, 0, 0)),
            pl.BlockSpec((_LPS, 1, _MAX_ATT), lambda g, o, c: (g, 0, 0)),
        ],
        out_specs=pl.BlockSpec((_KS, _MAX_ATT), lambda g, o, c: (0, 0)),
    )
    return pl.pallas_call(
        _heads_body,
        grid_spec=grid_spec,
        out_shape=jax.ShapeDtypeStruct((_KS, _MAX_ATT), jnp.float32),
    )(poffs, cats, h_sorted, W_heads,
      b_heads.reshape(_NUM_ATTR, 1, _MAX_ATT))


def _unsort_body(sposr_ref, ls_ref, out_ref):
    sel = (lax.broadcasted_iota(jnp.int32, (_K, _KS), 0) == sposr_ref[:]
           ).astype(jnp.float32)
    out_ref[:] = jnp.dot(sel, ls_ref[:], preferred_element_type=jnp.float32)


def _unsort(spos, logits_sorted):
    return pl.pallas_call(
        _unsort_body,
        out_shape=jax.ShapeDtypeStruct((_K, _MAX_ATT), jnp.float32),
    )(spos.reshape(1, _KS), logits_sorted)


def kernel(x, boxes, box_labels, W_ff, b_ff, ln_g, ln_b, W_disr, b_disr,
           W_heads, b_heads):
    q = jnp.round(boxes[:, 1:5].astype(jnp.float32) * _SCALE).astype(jnp.int32)
    x1, y1, x2, y2 = q[:, 0], q[:, 1], q[:, 2], q[:, 3]
    roi_w = jnp.maximum(x2 - x1 + 1, 1)
    roi_h = jnp.maximum(y2 - y1 + 1, 1)
    hs = jnp.clip(y1, 0, _SIDE)
    he = jnp.clip(y1 + roi_h, 0, _SIDE)
    ws = jnp.clip(x1, 0, _SIDE)
    we = jnp.clip(x1 + roi_w, 0, _SIDE)
    b = boxes[:, 0].astype(jnp.int32)
    nrows = he - hs
    ncols = we - ws
    base = b * (_SIDE * _SIDE) + hs * _SIDE + ws
    n = nrows * ncols
    magic = (65536 + jnp.maximum(ncols, 1) - 1) // jnp.maximum(ncols, 1)
    valid = (n > 0).astype(jnp.int32)
    n_eff = jnp.maximum(n, 1)
    ncol_eff = jnp.maximum(ncols, 1)
    base_eff = jnp.where(n > 0, base, 0)
    tchunks = jnp.repeat(
        ((n_eff + 63) // 64).reshape(32, 16).sum(axis=1).astype(jnp.int32),
        16)
    meta = jnp.stack(
        [base_eff, n_eff, ncol_eff, magic, valid, tchunks]
        + [jnp.zeros_like(n)] * 10, axis=1)

    pooled = _sc_pool_call(x.reshape(-1, _D), meta.reshape(-1))

    labels = box_labels.astype(jnp.int32)
    perm = jnp.argsort(labels).astype(jnp.int32)
    counts = jnp.zeros((_NUM_ATTR,), jnp.int32).at[labels].add(1)
    offs = jnp.concatenate([jnp.zeros((1,), jnp.int32),
                            jnp.cumsum(counts).astype(jnp.int32)])
    pcounts = ((counts + 7) // 8) * 8
    poffs = jnp.concatenate([jnp.zeros((1,), jnp.int32),
                             jnp.cumsum(pcounts).astype(jnp.int32)])
    cats = jnp.asarray(_ID2CAT, dtype=jnp.int32)
    labels_sorted = jnp.sort(labels)
    dest = poffs[labels_sorted] + jnp.arange(_K, dtype=jnp.int32) \
        - offs[labels_sorted]
    spos = jnp.full((_KS,), -1, jnp.int32).at[dest].set(perm)

    h, h_sorted, disr_logits = _ff(pooled, W_ff, b_ff, ln_g, ln_b,
                                   W_disr, b_disr, spos)
    logits_sorted = _heads(poffs, cats, h_sorted, W_heads, b_heads)
    logits = _unsort(spos, logits_sorted)
    return (h, logits, disr_logits)


# speedup vs baseline: 1.1701x; 1.0008x over previous
import functools

import jax
import jax.numpy as jnp
from jax import lax
from jax.experimental import pallas as pl
from jax.experimental.pallas import tpu as pltpu
from jax.experimental.pallas import tpu_sc as plsc

_ID2CAT = tuple(int(2 + (i * 97) % 398) for i in range(120))
_MAX_ATT = 397
_NUM_ATTR = 120
_D = 768
_K = 512
_SIDE = 32
_SCALE = 32.0 / 512.0


_D16 = _D // 16


_CH = 64
_NG = 4
_GV = _D16 // _NG


def _sc_pool_call(x2d, meta_flat):
    info = plsc.get_sparse_core_info()
    nc, ns = info.num_cores, info.num_subcores
    nw = nc * ns
    bpw = _K // nw
    mesh = plsc.VectorSubcoreMesh(core_axis_name="c", subcore_axis_name="s")

    @functools.partial(
        pl.kernel,
        mesh=mesh,
        out_type=jax.ShapeDtypeStruct((_K, _D), jnp.float32),
        scratch_types=[
            pltpu.VMEM((bpw * 16,), jnp.int32),
            pltpu.VMEM((_CH,), jnp.int32),
            pltpu.VMEM((_CH,), jnp.int32),
            pltpu.VMEM((_CH, _D), jnp.float32),
            pltpu.VMEM((_CH, _D), jnp.float32),
            pltpu.VMEM((_D,), jnp.float32),
            pltpu.SemaphoreType.DMA,
            pltpu.SemaphoreType.DMA,
        ],
    )
    def k(x_hbm, meta_hbm, out_hbm, meta_v, idx0, idx1, rows0, rows1,
          acc_v, sem0, sem1):
        wid = lax.axis_index("s") * nc + lax.axis_index("c")
        base_box = wid * bpw
        pltpu.sync_copy(meta_hbm.at[pl.ds(base_box * 16, bpw * 16)], meta_v)
        lanes = lax.iota(jnp.int32, 16)

        def fire(slot, chunk, idx_v, rows_v, sem):
            srow = meta_v[pl.ds(slot * 16, 16)]
            basef, nf, ncolf, magicf = srow[0], srow[1], srow[2], srow[3]
            for u in range(_CH // 16):
                t = jnp.minimum(chunk * _CH + u * 16 + lanes, nf - 1)
                yy = lax.shift_right_logical(t * magicf, 16)
                idx_v[pl.ds(u * 16, 16)] = basef + yy * _SIDE + (t - yy * ncolf)
            pltpu.async_copy(x_hbm.at[idx_v], rows_v, sem)

        def advance(slot, chunk):
            srow = meta_v[pl.ds(slot * 16, 16)]
            nch = lax.shift_right_logical(srow[1] + (_CH - 1), 6)
            over = (chunk + 1) >= nch
            return (jnp.where(over, slot + 1, slot),
                    jnp.where(over, 0, chunk + 1))

        def reduce_rows(rows_v, pc, nrows, last, valid, outrow):
            for g in range(_NG):
                first = pc == 0
                regs = [
                    jnp.where(first, jnp.full((16,), -jnp.inf, jnp.float32),
                              acc_v[pl.ds((g * _GV + v) * 16, 16)])
                    for v in range(_GV)
                ]

                def row_step(r4, regs):
                    for rr in range(4):
                        r = jnp.minimum(r4 * 4 + rr, nrows - 1)
                        regs = tuple(
                            jnp.maximum(
                                regs[v],
                                rows_v[r, pl.ds((g * _GV + v) * 16, 16)])
                            for v in range(_GV))
                    return regs

                regs = lax.fori_loop(0, (nrows + 3) // 4, row_step,
                                     tuple(regs))
                for v in range(_GV):
                    acc_v[pl.ds((g * _GV + v) * 16, 16)] = jnp.where(
                        valid > 0, regs[v], jnp.zeros((16,), jnp.float32))

            @pl.when(last)
            def _():
                pltpu.sync_copy(acc_v, out_hbm.at[outrow])

        tchunks = meta_v[pl.ds(0, 16)][5]
        fire(0, 0, idx0, rows0, sem0)
        fi0, fc0 = advance(0, 0)

        def body(q, carry):
            pi, pc, fi, fc = carry
            parity = jnp.bitwise_and(q, 1)

            @pl.when((fi < bpw) & (parity == 0))
            def _():
                fire(fi, fc, idx1, rows1, sem1)

            @pl.when((fi < bpw) & (parity == 1))
            def _():
                fire(fi, fc, idx0, rows0, sem0)

            prow = meta_v[pl.ds(pi * 16, 16)]
            np_ = prow[1]
            valid = prow[4]
            outrow = base_box + pi
            nchp = lax.shift_right_logical(np_ + (_CH - 1), 6)
            nrows = jnp.minimum(np_ - pc * _CH, _CH)
            last = (pc + 1) >= nchp

            @pl.when(parity == 0)
            def _():
                pltpu.make_async_copy(x_hbm.at[pl.ds(0, _CH)], rows0,
                                      sem0).wait()
                reduce_rows(rows0, pc, nrows, last, valid, outrow)

            @pl.when(parity == 1)
            def _():
                pltpu.make_async_copy(x_hbm.at[pl.ds(0, _CH)], rows1,
                                      sem1).wait()
                reduce_rows(rows1, pc, nrows, last, valid, outrow)

            fi2, fc2 = advance(fi, fc)
            fi3 = jnp.where(fi < bpw, fi2, fi)
            fc3 = jnp.where(fi < bpw, fc2, fc)
            pi2, pc2 = advance(pi, pc)
            return (pi2, pc2, fi3, fc3)

        lax.fori_loop(0, tchunks, body,
                      (jnp.int32(0), jnp.int32(0),
                       fi0.astype(jnp.int32), fc0.astype(jnp.int32)))

    return k(x2d, meta_flat)


_KS = 1360


def _ff_body(p_ref, wff_ref, bff_ref, g_ref, be_ref, wd_ref, bd_ref,
             perm_ref, h_ref, hs_ref, disr_ref):
    h0 = jnp.dot(p_ref[:], wff_ref[:], preferred_element_type=jnp.float32)
    h0 = h0 + bff_ref[:]
    mu = jnp.mean(h0, axis=-1, keepdims=True)
    var = jnp.mean((h0 - mu) ** 2, axis=-1, keepdims=True)
    hn = (h0 - mu) / jnp.sqrt(var + 1e-5) * g_ref[:] + be_ref[:]
    h = hn * 0.5 * (1.0 + lax.erf(hn / jnp.sqrt(jnp.float32(2.0))))
    h_ref[:] = h
    disr_ref[:] = jnp.dot(h, wd_ref[:], preferred_element_type=jnp.float32) + bd_ref[:]
    sel = (lax.broadcasted_iota(jnp.int32, (_KS, _K), 1) == perm_ref[:]
           ).astype(jnp.float32)
    hs_ref[:] = jnp.dot(sel, h, preferred_element_type=jnp.float32)


def _ff(pooled, W_ff, b_ff, ln_g, ln_b, W_disr, b_disr, spos):
    return pl.pallas_call(
        _ff_body,
        out_shape=(jax.ShapeDtypeStruct((_K, _D), jnp.float32),
                   jax.ShapeDtypeStruct((_KS, _D), jnp.float32),
                   jax.ShapeDtypeStruct((_K, 1), jnp.float32)),
    )(pooled, W_ff, b_ff.reshape(1, _D), ln_g.reshape(1, _D),
      ln_b.reshape(1, _D), W_disr, b_disr.reshape(1, 1),
      spos.reshape(_KS, 1))


_LPS = 8


def _heads_body(poffs_ref, cats_ref, hs_ref, w_ref, bh_ref, out_ref):
    g = pl.program_id(0)

    @pl.when(g == 0)
    def _():
        out_ref[...] = jnp.zeros((_KS, _MAX_ATT), jnp.float32)

    def one_chunk(el, o):
        cat = cats_ref[g * _LPS + el]
        colmask = lax.broadcasted_iota(jnp.int32, (8, _MAX_ATT), 1) < cat
        rows = hs_ref[pl.ds(o, 8), :]
        prod = jnp.dot(rows, w_ref[el], preferred_element_type=jnp.float32)
        out_ref[pl.ds(o, 8), :] = jnp.where(
            colmask, prod + bh_ref[el, 0], 0.0)

    for el in range(_LPS):
        one_chunk(el, pl.multiple_of(poffs_ref[g * _LPS + el], 8))

    for el in range(_LPS):
        e = g * _LPS + el
        start = pl.multiple_of(poffs_ref[e], 8)
        nch = (poffs_ref[e + 1] - poffs_ref[e]) // 8

        def chunk(c, carry):
            one_chunk(el, start + c * 8)
            return carry

        lax.fori_loop(1, nch, chunk, 0)


def _heads(poffs, cats, h_sorted, W_heads, b_heads):
    grid_spec = pltpu.PrefetchScalarGridSpec(
        num_scalar_prefetch=2,
        grid=(_NUM_ATTR // _LPS,),
        in_specs=[
            pl.BlockSpec((_KS, _D), lambda g, o, c: (0, 0)),
            pl.BlockSpec((_LPS, _D, _MAX_ATT), lambda g, o, c: (g, 0, 0)),
            pl.BlockSpec((_LPS, 1, _MAX_ATT), lambda g, o, c: (g, 0, 0)),
        ],
        out_specs=pl.BlockSpec((_KS, _MAX_ATT), lambda g, o, c: (0, 0)),
    )
    return pl.pallas_call(
        _heads_body,
        grid_spec=grid_spec,
        out_shape=jax.ShapeDtypeStruct((_KS, _MAX_ATT), jnp.float32),
    )(poffs, cats, h_sorted, W_heads,
      b_heads.reshape(_NUM_ATTR, 1, _MAX_ATT))


def _unsort_body(sposr_ref, ls_ref, out_ref):
    sel = (lax.broadcasted_iota(jnp.int32, (_K, _KS), 0) == sposr_ref[:]
           ).astype(jnp.float32)
    out_ref[:] = jnp.dot(sel, ls_ref[:], preferred_element_type=jnp.float32)


def _unsort(spos, logits_sorted):
    return pl.pallas_call(
        _unsort_body,
        out_shape=jax.ShapeDtypeStruct((_K, _MAX_ATT), jnp.float32),
    )(spos.reshape(1, _KS), logits_sorted)


def kernel(x, boxes, box_labels, W_ff, b_ff, ln_g, ln_b, W_disr, b_disr,
           W_heads, b_heads):
    q = jnp.round(boxes[:, 1:5].astype(jnp.float32) * _SCALE).astype(jnp.int32)
    x1, y1, x2, y2 = q[:, 0], q[:, 1], q[:, 2], q[:, 3]
    roi_w = jnp.maximum(x2 - x1 + 1, 1)
    roi_h = jnp.maximum(y2 - y1 + 1, 1)
    hs = jnp.clip(y1, 0, _SIDE)
    he = jnp.clip(y1 + roi_h, 0, _SIDE)
    ws = jnp.clip(x1, 0, _SIDE)
    we = jnp.clip(x1 + roi_w, 0, _SIDE)
    b = boxes[:, 0].astype(jnp.int32)
    nrows = he - hs
    ncols = we - ws
    base = b * (_SIDE * _SIDE) + hs * _SIDE + ws
    n = nrows * ncols
    magic = (65536 + jnp.maximum(ncols, 1) - 1) // jnp.maximum(ncols, 1)
    valid = (n > 0).astype(jnp.int32)
    n_eff = jnp.maximum(n, 1)
    ncol_eff = jnp.maximum(ncols, 1)
    base_eff = jnp.where(n > 0, base, 0)
    tchunks = jnp.repeat(
        ((n_eff + 63) // 64).reshape(32, 16).sum(axis=1).astype(jnp.int32),
        16)
    meta = jnp.stack(
        [base_eff, n_eff, ncol_eff, magic, valid, tchunks]
        + [jnp.zeros_like(n)] * 10, axis=1)

    pooled = _sc_pool_call(x.reshape(-1, _D), meta.reshape(-1))

    labels = box_labels.astype(jnp.int32)
    perm = jnp.argsort(labels).astype(jnp.int32)
    counts = jnp.zeros((_NUM_ATTR,), jnp.int32).at[labels].add(1)
    offs = jnp.concatenate([jnp.zeros((1,), jnp.int32),
                            jnp.cumsum(counts).astype(jnp.int32)])
    pcounts = ((counts + 7) // 8) * 8
    poffs = jnp.concatenate([jnp.zeros((1,), jnp.int32),
                             jnp.cumsum(pcounts).astype(jnp.int32)])
    cats = jnp.asarray(_ID2CAT, dtype=jnp.int32)
    labels_sorted = jnp.sort(labels)
    dest = poffs[labels_sorted] + jnp.arange(_K, dtype=jnp.int32) \
        - offs[labels_sorted]
    spos = jnp.full((_KS,), -1, jnp.int32).at[dest].set(perm)

    h, h_sorted, disr_logits = _ff(pooled, W_ff, b_ff, ln_g, ln_b,
                                   W_disr, b_disr, spos)
    logits_sorted = _heads(poffs, cats, h_sorted, W_heads, b_heads)
    logits = _unsort(spos, logits_sorted)
    return (h, logits, disr_logits)
